# baseline (speedup 1.0000x reference)
import functools

import jax
import jax.numpy as jnp
from jax import lax
from jax.experimental import pallas as pl
from jax.experimental.pallas import tpu as pltpu
from jax.experimental.pallas import tpu_sc as plsc

_B = 16384
_C = 26
_V = 100000
_D = 32
_N = _B * _C
_NC = 2
_NS = 16
_NW = _NC * _NS
_RPW = _N // _NW
_IPD = 128
_DMAS = 4
_CROWS = _DMAS * _IPD
_NCHUNK = _RPW // _CROWS

_mesh = plsc.VectorSubcoreMesh(core_axis_name="c", subcore_axis_name="s")


@functools.partial(
    pl.kernel,
    mesh=_mesh,
    compiler_params=pltpu.CompilerParams(use_tc_tiling_on_sc=False),
    out_type=jax.ShapeDtypeStruct((_N // _IPD, _IPD, _D), jnp.float32),
    scratch_types=[
        pltpu.VMEM((_RPW,), jnp.int32),
        pltpu.VMEM((2, _DMAS, _IPD, _D), jnp.float32),
        pltpu.SemaphoreType.DMA,
        pltpu.SemaphoreType.DMA,
    ],
)
def _encoder(idx_hbm, tab_hbm, out_hbm, idx_v, rows_v, gsem, ssem):
    wid = lax.axis_index("s") * _NC + lax.axis_index("c")
    base = wid * _RPW

    pltpu.sync_copy(idx_hbm.at[pl.ds(base, _RPW)], idx_v)

    offs0 = lax.iota(jnp.int32, 16) * _V

    def adj(t, offs):
        sl = pl.ds(t * 16, 16)
        idx_v[sl] = idx_v[sl] + offs
        offs = offs + 16 * _V
        return jnp.where(offs >= _C * _V, offs - _C * _V, offs)

    lax.fori_loop(0, _RPW // 16, adj, offs0)

    out_base = wid * (_RPW // _IPD)

    def chunk_body(ci, carry):
        copies = []
        for d in range(_DMAS):
            copies.append(pltpu.async_copy(
                tab_hbm.at[idx_v.at[pl.ds((ci * _DMAS + d) * _IPD, _IPD)]],
                rows_v.at[0, d],
                gsem,
            ))
        for cp in copies:
            cp.wait()
        pltpu.sync_copy(rows_v.at[0],
                        out_hbm.at[pl.ds(out_base + ci * _DMAS, _DMAS)])
        return carry

    lax.fori_loop(0, _NCHUNK, chunk_body, 0)


def kernel(tensor, tables):
    idx = tensor.reshape(_N).astype(jnp.int32)
    tab = tables.reshape(_C * _V, _D)
    out = _encoder(idx, tab)
    return out.reshape(_B, _C * _D)


# speedup vs baseline: 2.4279x; 2.4279x over previous
import functools

import jax
import jax.numpy as jnp
from jax import lax
from jax.experimental import pallas as pl
from jax.experimental.pallas import tpu as pltpu
from jax.experimental.pallas import tpu_sc as plsc

_B = 16384
_C = 26
_V = 100000
_D = 32
_NC = 2
_NS = 16
_CHUNK = 2048

_mesh = plsc.VectorSubcoreMesh(core_axis_name="c", subcore_axis_name="s")


@functools.partial(
    pl.kernel,
    mesh=_mesh,
    compiler_params=pltpu.CompilerParams(
        use_tc_tiling_on_sc=True, needs_layout_passes=False),
    out_type=jax.ShapeDtypeStruct((_C * _D, _B), jnp.float32),
    scratch_types=[
        pltpu.VMEM((_V,), jnp.float32),
        pltpu.VMEM((_CHUNK,), jnp.int32),
        pltpu.VMEM((_CHUNK,), jnp.float32),
    ],
)
def _encoder(idx_hbm, tab_hbm, out_hbm, col_v, idx_v, res_v):
    d = lax.axis_index("c") * _NS + lax.axis_index("s")

    def col_body(c, carry):
        pltpu.sync_copy(tab_hbm.at[c, d], col_v)

        def chunk_body(k, carry2):
            b0 = k * _CHUNK
            pltpu.sync_copy(idx_hbm.at[pl.ds(c * _B + b0, _CHUNK)], idx_v)

            def gather(j, carry3):
                iv = idx_v[pl.ds(j * 16, 16)]
                res_v[pl.ds(j * 16, 16)] = plsc.load_gather(col_v, [iv])
                return carry3

            lax.fori_loop(0, _CHUNK // 16, gather, 0, unroll=8)
            pltpu.sync_copy(res_v, out_hbm.at[c * _D + d, pl.ds(b0, _CHUNK)])
            return carry2

        lax.fori_loop(0, _B // _CHUNK, chunk_body, 0)
        return carry

    lax.fori_loop(0, _C, col_body, 0)


def kernel(tensor, tables):
    idx = tensor.T.reshape(_C * _B).astype(jnp.int32)
    tab = jnp.transpose(tables, (0, 2, 1))
    out_t = _encoder(idx, tab)
    return out_t.T.reshape(_B, _C * _D)


# speedup vs baseline: 3.4089x; 1.4041x over previous
import functools

import jax
import jax.numpy as jnp
from jax import lax
from jax.experimental import pallas as pl
from jax.experimental.pallas import tpu as pltpu
from jax.experimental.pallas import tpu_sc as plsc

_B = 16384
_C = 26
_V = 100000
_D = 32
_NC = 2
_NS = 16
_CHUNK = 2048
_NK = _B // _CHUNK

_mesh = plsc.VectorSubcoreMesh(core_axis_name="c", subcore_axis_name="s")


@functools.partial(
    pl.kernel,
    mesh=_mesh,
    compiler_params=pltpu.CompilerParams(
        use_tc_tiling_on_sc=True, needs_layout_passes=False),
    out_type=jax.ShapeDtypeStruct((_C * _D, _B), jnp.float32),
    scratch_types=[
        pltpu.VMEM((_V,), jnp.float32),
        pltpu.VMEM((2, _CHUNK), jnp.int32),
        pltpu.VMEM((2, _CHUNK), jnp.float32),
        pltpu.SemaphoreType.DMA,
        pltpu.SemaphoreType.DMA,
        pltpu.SemaphoreType.DMA,
    ],
)
def _encoder(idx_hbm, tab_hbm, out_hbm, col_v, idx_v, res_v,
             colsem, idxsem, ssem):
    d = lax.axis_index("c") * _NS + lax.axis_index("s")

    def idx_load(c, k, slot):
        pltpu.async_copy(
            idx_hbm.at[pl.ds(c * _B + k * _CHUNK, _CHUNK)],
            idx_v.at[slot], idxsem)

    def idx_wait(slot):
        pltpu.make_async_copy(
            idx_hbm.at[pl.ds(0, _CHUNK)], idx_v.at[slot], idxsem).wait()

    def res_wait(slot):
        pltpu.make_async_copy(
            res_v.at[slot], out_hbm.at[0, pl.ds(0, _CHUNK)], ssem).wait()

    pltpu.async_copy(tab_hbm.at[0, d], col_v, colsem)
    idx_load(0, 0, 0)
    idx_load(0, 1, 1)

    def col_body(c, carry):
        pltpu.make_async_copy(tab_hbm.at[c, d], col_v, colsem).wait()
        f = c * _D + d
        for k in range(_NK):
            slot = k % 2
            idx_wait(slot)
            if k >= 2:
                res_wait(slot)
            else:
                @pl.when(c >= 1)
                def _():
                    res_wait(slot)

            def gather(j, carry2):
                iv = idx_v[slot, pl.ds(j * 16, 16)]
                res_v[slot, pl.ds(j * 16, 16)] = plsc.load_gather(col_v, [iv])
                return carry2

            lax.fori_loop(0, _CHUNK // 16, gather, 0, unroll=8)
            pltpu.async_copy(
                res_v.at[slot], out_hbm.at[f, pl.ds(k * _CHUNK, _CHUNK)], ssem)
            if k < _NK - 2:
                idx_load(c, k + 2, slot)
            else:
                @pl.when(c + 1 < _C)
                def _():
                    idx_load(c + 1, k + 2 - _NK, slot)
        @pl.when(c + 1 < _C)
        def _():
            pltpu.async_copy(tab_hbm.at[c + 1, d], col_v, colsem)
        return carry

    lax.fori_loop(0, _C, col_body, 0)
    res_wait(0)
    res_wait(1)


def kernel(tensor, tables):
    idx = tensor.T.reshape(_C * _B).astype(jnp.int32)
    tab = jnp.transpose(tables, (0, 2, 1))
    out_t = _encoder(idx, tab)
    return out_t.T.reshape(_B, _C * _D)


# speedup vs baseline: 5.4260x; 1.5917x over previous
import functools

import jax
import jax.numpy as jnp
from jax import lax
from jax.experimental import pallas as pl
from jax.experimental.pallas import tpu as pltpu
from jax.experimental.pallas import tpu_sc as plsc

_B = 16384
_C = 26
_V = 100000
_D = 32
_NC = 2
_NS = 16
_CHUNK = 2048
_NK = _B // _CHUNK

_mesh = plsc.VectorSubcoreMesh(core_axis_name="c", subcore_axis_name="s")


@functools.partial(
    pl.kernel,
    mesh=_mesh,
    compiler_params=pltpu.CompilerParams(
        use_tc_tiling_on_sc=True, needs_layout_passes=False),
    out_type=jax.ShapeDtypeStruct((_C * _D, _B), jnp.float32),
    scratch_types=[
        pltpu.VMEM((_V,), jnp.float32),
        pltpu.VMEM((2, _CHUNK), jnp.int32),
        pltpu.VMEM((2, _CHUNK), jnp.float32),
        pltpu.SemaphoreType.DMA,
        pltpu.SemaphoreType.DMA,
        pltpu.SemaphoreType.DMA,
    ],
)
def _encoder(idx_hbm, tab_hbm, out_hbm, col_v, idx_v, res_v,
             colsem, idxsem, ssem):
    d = lax.axis_index("c") * _NS + lax.axis_index("s")

    def idx_load(c, k, slot):
        pltpu.async_copy(
            idx_hbm.at[pl.ds(c * _B + k * _CHUNK, _CHUNK)],
            idx_v.at[slot], idxsem)

    def idx_wait(slot):
        pltpu.make_async_copy(
            idx_hbm.at[pl.ds(0, _CHUNK)], idx_v.at[slot], idxsem).wait()

    def res_wait(slot):
        pltpu.make_async_copy(
            res_v.at[slot], out_hbm.at[0, pl.ds(0, _CHUNK)], ssem).wait()

    pltpu.async_copy(tab_hbm.at[0, d], col_v, colsem)
    idx_load(0, 0, 0)
    idx_load(0, 1, 1)

    def col_body(c, carry):
        pltpu.make_async_copy(tab_hbm.at[c, d], col_v, colsem).wait()
        f = c * _D + d
        for k in range(_NK):
            slot = k % 2
            idx_wait(slot)
            if k >= 2:
                res_wait(slot)
            else:
                @pl.when(c >= 1)
                def _():
                    res_wait(slot)

            @plsc.parallel_loop(0, _CHUNK // 16, unroll=8)
            def _gather(j):
                iv = idx_v[slot, pl.ds(j * 16, 16)]
                res_v[slot, pl.ds(j * 16, 16)] = plsc.load_gather(col_v, [iv])
            pltpu.async_copy(
                res_v.at[slot], out_hbm.at[f, pl.ds(k * _CHUNK, _CHUNK)], ssem)
            if k < _NK - 2:
                idx_load(c, k + 2, slot)
            else:
                @pl.when(c + 1 < _C)
                def _():
                    idx_load(c + 1, k + 2 - _NK, slot)
        @pl.when(c + 1 < _C)
        def _():
            pltpu.async_copy(tab_hbm.at[c + 1, d], col_v, colsem)
        return carry

    lax.fori_loop(0, _C, col_body, 0)
    res_wait(0)
    res_wait(1)


def kernel(tensor, tables):
    idx = tensor.T.reshape(_C * _B).astype(jnp.int32)
    tab = jnp.transpose(tables, (0, 2, 1))
    out_t = _encoder(idx, tab)
    return out_t.T.reshape(_B, _C * _D)


# speedup vs baseline: 6.1261x; 1.1290x over previous
import functools

import jax
import jax.numpy as jnp
from jax import lax
from jax.experimental import pallas as pl
from jax.experimental.pallas import tpu as pltpu
from jax.experimental.pallas import tpu_sc as plsc

_B = 16384
_C = 26
_V = 100000
_D = 32
_NC = 2
_NS = 16
_CHUNK = 2048
_NK = _B // _CHUNK
_OFF = (0, 33280, 66560)
_LEN = (33280, 33280, 33408)
_TPAD = 128
_BUF = 33408 + _TPAD
_L2 = 33440

_mesh = plsc.VectorSubcoreMesh(core_axis_name="c", subcore_axis_name="s")


@functools.partial(
    pl.kernel,
    mesh=_mesh,
    compiler_params=pltpu.CompilerParams(
        use_tc_tiling_on_sc=True, needs_layout_passes=False),
    out_type=jax.ShapeDtypeStruct((_C * _D, _B), jnp.float32),
    scratch_types=[
        pltpu.VMEM((_BUF,), jnp.float32),
        pltpu.VMEM((_BUF,), jnp.float32),
        pltpu.VMEM((_B,), jnp.int32),
        pltpu.VMEM((_B,), jnp.float32),
        pltpu.VMEM((2, _CHUNK), jnp.float32),
        pltpu.SemaphoreType.DMA,
        pltpu.SemaphoreType.DMA,
        pltpu.SemaphoreType.DMA,
    ],
)
def _encoder(idx_hbm, tab_hbm, tail_hbm, out_hbm, col_b0, col_b1, idx_v,
             res_v, obuf, colsem, idxsem, ssem):
    d = lax.axis_index("c") * _NS + lax.axis_index("s")
    col_bufs = (col_b0, col_b1)

    def t_issue(c, t, b):
        pltpu.async_copy(
            tab_hbm.at[c * _D + d, pl.ds(_OFF[t], _LEN[t])],
            col_bufs[b].at[pl.ds(0, _LEN[t])], colsem)
        if t == 2:
            pltpu.async_copy(
                tail_hbm.at[c, d],
                col_bufs[b].at[pl.ds(_LEN[2], _TPAD)], colsem)

    def t_wait(t, b):
        pltpu.make_async_copy(
            tab_hbm.at[0, pl.ds(0, _LEN[t])],
            col_bufs[b].at[pl.ds(0, _LEN[t])], colsem).wait()
        if t == 2:
            pltpu.make_async_copy(
                tail_hbm.at[0, 0],
                col_bufs[b].at[pl.ds(_LEN[2], _TPAD)], colsem).wait()

    def idx_refill(c, k):
        pltpu.async_copy(
            idx_hbm.at[c, pl.ds(k * _CHUNK, _CHUNK)],
            idx_v.at[pl.ds(k * _CHUNK, _CHUNK)], idxsem)

    def idx_wait():
        pltpu.make_async_copy(
            idx_hbm.at[0, pl.ds(0, _CHUNK)],
            idx_v.at[pl.ds(0, _CHUNK)], idxsem).wait()

    def store_wait():
        pltpu.make_async_copy(
            obuf.at[0], out_hbm.at[0, pl.ds(0, _CHUNK)], ssem).wait()

    def masked_vals(j, t, b):
        sl = pl.ds(j * 16, 16)
        iv = idx_v[sl]
        ivq = iv - _OFF[t]
        ext = _L2 if t == 2 else _LEN[t]
        m = plsc.bitcast(ivq, jnp.uint32) < jnp.uint32(ext)
        vals = plsc.load_gather(col_bufs[b], [ivq], mask=m)
        return sl, jnp.where(m, vals, jnp.float32(0.0))

    for k in range(_NK):
        idx_refill(0, k)
    t_issue(0, 0, 0)
    t_issue(0, 1, 1)

    def pair_body(i, carry):
        for p in range(2):
            c = 2 * i + p
            a, b = p, 1 - p
            for _k in range(_NK):
                idx_wait()
            t_wait(0, a)

            @plsc.parallel_loop(0, _B // 16, unroll=8)
            def _pass0(j):
                sl, vz = masked_vals(j, 0, a)
                res_v[sl] = vz

            t_issue(c, 2, a)
            t_wait(1, b)

            @plsc.parallel_loop(0, _B // 16, unroll=8)
            def _pass1(j):
                sl, vz = masked_vals(j, 1, b)
                plsc.addupdate(res_v.at[sl], vz)

            @pl.when(c + 1 < _C)
            def _():
                t_issue(c + 1, 0, b)
            t_wait(2, a)

            f = c * _D + d
            for k in range(_NK):
                oslot = k % 2
                if k >= 2:
                    store_wait()
                else:
                    @pl.when(c >= 1)
                    def _():
                        store_wait()

                @plsc.parallel_loop(k * (_CHUNK // 16), (k + 1) * (_CHUNK // 16),
                                    unroll=8)
                def _pass2(j):
                    sl, vz = masked_vals(j, 2, a)
                    lo = j * 16 - k * _CHUNK
                    obuf[oslot, pl.ds(lo, 16)] = res_v[sl] + vz

                pltpu.async_copy(
                    obuf.at[oslot],
                    out_hbm.at[f, pl.ds(k * _CHUNK, _CHUNK)], ssem)

                @pl.when(c + 1 < _C)
                def _():
                    idx_refill(c + 1, k)

            @pl.when(c + 1 < _C)
            def _():
                t_issue(c + 1, 1, a)
        return carry

    lax.fori_loop(0, _C // 2, pair_body, 0)
    store_wait()
    store_wait()


def kernel(tensor, tables):
    idx = tensor.T.astype(jnp.int32)
    tab = jnp.transpose(tables, (0, 2, 1))
    tail = jnp.pad(tab[:, :, _OFF[2] + _LEN[2]:], ((0, 0), (0, 0), (0, 96)))
    tab2 = tab.reshape(_C * _D, _V)
    out_t = _encoder(idx, tab2, tail)
    return out_t.T.reshape(_B, _C * _D)
